# baseline (speedup 1.0000x reference)
import functools

import jax
import jax.numpy as jnp
from jax import lax
from jax.experimental import pallas as pl
from jax.experimental.pallas import tpu as pltpu
from jax.experimental.pallas import tpu_sc as plsc

N_NODES = 10000
IN_DIM = 128
HID = 64
OUT_DIM = 128

NC = 2
NS = 16
L = 16
NW = NC * NS
CHUNK = 128
ROWS_PER_TILE = 640
NROWS = NS * ROWS_PER_TILE
DUMMY = N_NODES
DEGW = 16


def _mesh():
    return plsc.VectorSubcoreMesh(core_axis_name="c", subcore_axis_name="s")


@functools.lru_cache(maxsize=None)
def _deg_call(k_chunks):
    @functools.partial(
        pl.kernel,
        out_type=jax.ShapeDtypeStruct((NC, NROWS, DEGW), jnp.float32),
        mesh=_mesh(),
        scratch_types=[
            pltpu.VMEM((k_chunks, CHUNK), jnp.int32),
            pltpu.VMEM((CHUNK, DEGW), jnp.float32),
            pltpu.VMEM((CHUNK, DEGW), jnp.float32),
            pltpu.VMEM_SHARED((NROWS, DEGW), jnp.float32),
        ],
        compiler_params=pltpu.CompilerParams(use_tc_tiling_on_sc=False),
    )
    def deg_kernel(cols_hbm, out_hbm, colbuf, zbuf, obuf, hist):
        cid = lax.axis_index("c")
        sid = lax.axis_index("s")
        wid = sid * NC + cid

        def fill(i, _):
            zbuf[i, :] = jnp.zeros((L,), jnp.float32)
            obuf[i, :] = jnp.full((L,), 1.0, jnp.float32)
            return 0

        lax.fori_loop(0, CHUNK, fill, 0)

        for b in range(ROWS_PER_TILE // CHUNK):
            pltpu.sync_copy(zbuf, hist.at[pl.ds(sid * ROWS_PER_TILE + b * CHUNK, CHUNK)])
        plsc.subcore_barrier()

        pltpu.sync_copy(cols_hbm.at[wid], colbuf)

        def step(j, _):
            pltpu.sync_copy(obuf, hist.at[colbuf.at[j]], add=True)
            return 0

        lax.fori_loop(0, k_chunks, step, 0)
        plsc.subcore_barrier()

        for b in range(ROWS_PER_TILE // CHUNK):
            r0 = sid * ROWS_PER_TILE + b * CHUNK
            pltpu.sync_copy(hist.at[pl.ds(r0, CHUNK)], zbuf)
            pltpu.sync_copy(zbuf, out_hbm.at[cid, pl.ds(r0, CHUNK)])

    return deg_kernel


NBUF = 4


@functools.lru_cache(maxsize=None)
def _accum_call(d, k_chunks):
    assert k_chunks % NBUF == 0

    @functools.partial(
        pl.kernel,
        out_type=jax.ShapeDtypeStruct((NC, NROWS, d), jnp.float32),
        mesh=_mesh(),
        scratch_types=[
            pltpu.VMEM((k_chunks, CHUNK), jnp.int32),
            pltpu.VMEM((k_chunks, CHUNK), jnp.int32),
            [pltpu.VMEM((CHUNK, d), jnp.float32) for _ in range(NBUF)],
            pltpu.SemaphoreType.DMA((NBUF,)),
            pltpu.VMEM_SHARED((NROWS, d), jnp.float32),
        ],
        compiler_params=pltpu.CompilerParams(use_tc_tiling_on_sc=False),
    )
    def accum_kernel(g_hbm, rows_hbm, cols_hbm, out_hbm, rowbuf, colbuf, bufs, sem, accum):
        cid = lax.axis_index("c")
        sid = lax.axis_index("s")
        wid = sid * NC + cid
        buf0 = bufs[0]

        def fill(i, _):
            for k in range(d // L):
                buf0[i, pl.ds(k * L, L)] = jnp.zeros((L,), jnp.float32)
            return 0

        lax.fori_loop(0, CHUNK, fill, 0)

        for b in range(ROWS_PER_TILE // CHUNK):
            pltpu.sync_copy(buf0, accum.at[pl.ds(sid * ROWS_PER_TILE + b * CHUNK, CHUNK)])
        plsc.subcore_barrier()

        pltpu.sync_copy(rows_hbm.at[wid], rowbuf)
        pltpu.sync_copy(cols_hbm.at[wid], colbuf)

        for b in range(NBUF):
            pltpu.async_copy(g_hbm.at[rowbuf.at[b]], bufs[b], sem.at[b])

        def step(t, _):
            for b in range(NBUF):
                j = t * NBUF + b
                pltpu.make_async_copy(g_hbm.at[rowbuf.at[j]], bufs[b], sem.at[b]).wait()
                pltpu.sync_copy(bufs[b], accum.at[colbuf.at[j]], add=True)

                @pl.when(j + NBUF < k_chunks)
                def _():
                    pltpu.async_copy(
                        g_hbm.at[rowbuf.at[j + NBUF]], bufs[b], sem.at[b])
            return 0

        lax.fori_loop(0, k_chunks // NBUF, step, 0)
        plsc.subcore_barrier()

        for b in range(ROWS_PER_TILE // CHUNK):
            r0 = sid * ROWS_PER_TILE + b * CHUNK
            pltpu.sync_copy(accum.at[pl.ds(r0, CHUNK)], buf0)
            pltpu.sync_copy(buf0, out_hbm.at[cid, pl.ds(r0, CHUNK)])

    return accum_kernel


def _dinv_from_hist(hist):
    deg = hist[0, :N_NODES, 0] + hist[1, :N_NODES, 0] + 1.0
    return lax.rsqrt(deg)


def _t1_body(hist_ref, x_ref, w1_ref, g1_ref):
    dinv = _dinv_from_hist(hist_ref[...])
    h = jnp.dot(x_ref[...], w1_ref[...].T, preferred_element_type=jnp.float32)
    g1_ref[...] = h * dinv[:, None]


def _t2_body(hist_ref, p_ref, g1_ref, b1_ref, gamma_ref, beta_ref, w2_ref, g2_ref):
    dinv = _dinv_from_hist(hist_ref[...])
    p = p_ref[...]
    a = (p[0, :N_NODES] + p[1, :N_NODES] + g1_ref[...]) * dinv[:, None] + b1_ref[...]
    mean = jnp.mean(a, axis=0)
    var = jnp.mean((a - mean) ** 2, axis=0)
    bn = (a - mean) * lax.rsqrt(var + 1e-5) * gamma_ref[...] + beta_ref[...]
    r = jnp.maximum(bn, 0.0)
    h2 = jnp.dot(r, w2_ref[...].T, preferred_element_type=jnp.float32)
    g2 = h2 * dinv[:, None]
    g2_ref[0] = g2[:, :HID]
    g2_ref[1] = g2[:, HID:]


def _t3_body(hist_ref, qa_ref, qb_ref, g2_ref, b2_ref, out_ref):
    dinv = _dinv_from_hist(hist_ref[...])
    qa = qa_ref[...]
    qb = qb_ref[...]
    g2 = g2_ref[...]
    oa = (qa[0, :N_NODES] + qa[1, :N_NODES] + g2[0]) * dinv[:, None]
    ob = (qb[0, :N_NODES] + qb[1, :N_NODES] + g2[1]) * dinv[:, None]
    out_ref[...] = jnp.concatenate([oa, ob], axis=1) + b2_ref[...]


def kernel(x, edge_index, W1, b1, gamma, beta, W2, b2):
    e = edge_index.shape[1]
    k_chunks = -(-e // (NW * CHUNK))
    k_chunks += (-k_chunks) % NBUF
    e_pad = k_chunks * NW * CHUNK
    pad = e_pad - e
    dummy_cols = DUMMY + (jnp.arange(pad, dtype=jnp.int32) % (NROWS - N_NODES))
    rows = jnp.concatenate(
        [edge_index[0], jnp.zeros((pad,), jnp.int32)]).reshape(NW, k_chunks, CHUNK)
    cols = jnp.concatenate(
        [edge_index[1], dummy_cols]).reshape(NW, k_chunks, CHUNK)

    hist = _deg_call(k_chunks)(cols)

    g1 = pl.pallas_call(
        _t1_body,
        out_shape=jax.ShapeDtypeStruct((N_NODES, HID), jnp.float32),
    )(hist, x, W1)

    p1 = _accum_call(HID, k_chunks)(g1, rows, cols)

    g2 = pl.pallas_call(
        _t2_body,
        out_shape=jax.ShapeDtypeStruct((2, N_NODES, HID), jnp.float32),
    )(hist, p1, g1, b1.reshape(1, HID), gamma.reshape(1, HID),
      beta.reshape(1, HID), W2)

    p2a = _accum_call(HID, k_chunks)(g2[0], rows, cols)
    p2b = _accum_call(HID, k_chunks)(g2[1], rows, cols)

    out = pl.pallas_call(
        _t3_body,
        out_shape=jax.ShapeDtypeStruct((N_NODES, OUT_DIM), jnp.float32),
    )(hist, p2a, p2b, g2, b2.reshape(1, OUT_DIM))

    return out


# speedup vs baseline: 1.1202x; 1.1202x over previous
import functools

import jax
import jax.numpy as jnp
from jax import lax
from jax.experimental import pallas as pl
from jax.experimental.pallas import tpu as pltpu
from jax.experimental.pallas import tpu_sc as plsc

N_NODES = 10000
IN_DIM = 128
HID = 64
OUT_DIM = 128

NC = 2
NS = 16
L = 16
CHUNK = 128
NBUF = 4
ROWS_PER_TILE = 640
NROWS = NS * ROWS_PER_TILE
DUMMY = N_NODES
DEGW = 16
CORE_RATIO = 4

_params = pltpu.CompilerParams(use_tc_tiling_on_sc=False)


def _mesh():
    return plsc.VectorSubcoreMesh(core_axis_name="c", subcore_axis_name="s")


def _splits(e):
    total = -(-e // CHUNK)
    t0 = -(-total // NS)
    k1 = max(NBUF, ((t0 + CORE_RATIO) // (CORE_RATIO + 1) + NBUF - 1) // NBUF * NBUF)
    k0 = (max(t0 - k1, NBUF) + NBUF - 1) // NBUF * NBUF
    return k0, k1


def _count(cid, k0, k1):
    return jnp.where(cid == 0, k0, k1)


@functools.lru_cache(maxsize=None)
def _deg_call(k0, k1):
    @functools.partial(
        pl.kernel,
        out_type=jax.ShapeDtypeStruct((NC, NROWS, DEGW), jnp.float32),
        mesh=_mesh(),
        scratch_types=[
            pltpu.VMEM((k0, CHUNK), jnp.int32),
            pltpu.VMEM((CHUNK, DEGW), jnp.float32),
            pltpu.VMEM((CHUNK, DEGW), jnp.float32),
            pltpu.VMEM_SHARED((NROWS, DEGW), jnp.float32),
        ],
        compiler_params=_params,
    )
    def deg_kernel(cols0_hbm, cols1_hbm, out_hbm, colbuf, zbuf, obuf, hist):
        cid = lax.axis_index("c")
        sid = lax.axis_index("s")

        def fill(i, _):
            zbuf[i, :] = jnp.zeros((L,), jnp.float32)
            obuf[i, :] = jnp.full((L,), 1.0, jnp.float32)
            return 0

        lax.fori_loop(0, CHUNK, fill, 0)

        for b in range(ROWS_PER_TILE // CHUNK):
            pltpu.sync_copy(zbuf, hist.at[pl.ds(sid * ROWS_PER_TILE + b * CHUNK, CHUNK)])
        plsc.subcore_barrier()

        @pl.when(cid == 0)
        def _():
            pltpu.sync_copy(cols0_hbm.at[sid], colbuf.at[pl.ds(0, k0)])

        @pl.when(cid == 1)
        def _():
            pltpu.sync_copy(cols1_hbm.at[sid], colbuf.at[pl.ds(0, k1)])

        def step(j, _):
            pltpu.sync_copy(obuf, hist.at[colbuf.at[j]], add=True)
            return 0

        lax.fori_loop(0, _count(cid, k0, k1), step, 0)
        plsc.subcore_barrier()

        for b in range(ROWS_PER_TILE // CHUNK):
            r0 = sid * ROWS_PER_TILE + b * CHUNK
            pltpu.sync_copy(hist.at[pl.ds(r0, CHUNK)], zbuf)
            pltpu.sync_copy(zbuf, out_hbm.at[cid, pl.ds(r0, CHUNK)])

    return deg_kernel


@functools.lru_cache(maxsize=None)
def _accum_call(d, k0, k1):
    @functools.partial(
        pl.kernel,
        out_type=jax.ShapeDtypeStruct((NC, NROWS, d), jnp.float32),
        mesh=_mesh(),
        scratch_types=[
            pltpu.VMEM((k0, CHUNK), jnp.int32),
            pltpu.VMEM((k0, CHUNK), jnp.int32),
            [pltpu.VMEM((CHUNK, d), jnp.float32) for _ in range(NBUF)],
            pltpu.SemaphoreType.DMA((NBUF,)),
            pltpu.VMEM_SHARED((NROWS, d), jnp.float32),
        ],
        compiler_params=_params,
    )
    def accum_kernel(g_hbm, rows0_hbm, cols0_hbm, rows1_hbm, cols1_hbm, out_hbm,
                     rowbuf, colbuf, bufs, sem, accum):
        cid = lax.axis_index("c")
        sid = lax.axis_index("s")
        count = _count(cid, k0, k1)
        buf0 = bufs[0]

        def fill(i, _):
            for k in range(d // L):
                buf0[i, pl.ds(k * L, L)] = jnp.zeros((L,), jnp.float32)
            return 0

        lax.fori_loop(0, CHUNK, fill, 0)

        for b in range(ROWS_PER_TILE // CHUNK):
            pltpu.sync_copy(buf0, accum.at[pl.ds(sid * ROWS_PER_TILE + b * CHUNK, CHUNK)])
        plsc.subcore_barrier()

        @pl.when(cid == 0)
        def _():
            pltpu.sync_copy(rows0_hbm.at[sid], rowbuf.at[pl.ds(0, k0)])
            pltpu.sync_copy(cols0_hbm.at[sid], colbuf.at[pl.ds(0, k0)])

        @pl.when(cid == 1)
        def _():
            pltpu.sync_copy(rows1_hbm.at[sid], rowbuf.at[pl.ds(0, k1)])
            pltpu.sync_copy(cols1_hbm.at[sid], colbuf.at[pl.ds(0, k1)])

        for b in range(NBUF):
            pltpu.async_copy(g_hbm.at[rowbuf.at[b]], bufs[b], sem.at[b])

        def step(t, _):
            for b in range(NBUF):
                j = t * NBUF + b
                pltpu.make_async_copy(g_hbm.at[rowbuf.at[j]], bufs[b], sem.at[b]).wait()
                pltpu.sync_copy(bufs[b], accum.at[colbuf.at[j]], add=True)

                @pl.when(j + NBUF < count)
                def _():
                    pltpu.async_copy(
                        g_hbm.at[rowbuf.at[j + NBUF]], bufs[b], sem.at[b])
            return 0

        lax.fori_loop(0, count // NBUF, step, 0)
        plsc.subcore_barrier()

        for b in range(ROWS_PER_TILE // CHUNK):
            r0 = sid * ROWS_PER_TILE + b * CHUNK
            pltpu.sync_copy(accum.at[pl.ds(r0, CHUNK)], buf0)
            pltpu.sync_copy(buf0, out_hbm.at[cid, pl.ds(r0, CHUNK)])

    return accum_kernel


def _dinv_from_hist(hist):
    deg = hist[0, :N_NODES, 0] + hist[1, :N_NODES, 0] + 1.0
    return lax.rsqrt(deg)


def _t1_body(hist_ref, x_ref, w1_ref, g1_ref):
    dinv = _dinv_from_hist(hist_ref[...])
    h = jnp.dot(x_ref[...], w1_ref[...].T, preferred_element_type=jnp.float32)
    g1_ref[...] = h * dinv[:, None]


def _t2_body(hist_ref, p_ref, g1_ref, b1_ref, gamma_ref, beta_ref, w2_ref, g2_ref):
    dinv = _dinv_from_hist(hist_ref[...])
    p = p_ref[...]
    a = (p[0, :N_NODES] + p[1, :N_NODES] + g1_ref[...]) * dinv[:, None] + b1_ref[...]
    mean = jnp.mean(a, axis=0)
    var = jnp.mean((a - mean) ** 2, axis=0)
    bn = (a - mean) * lax.rsqrt(var + 1e-5) * gamma_ref[...] + beta_ref[...]
    r = jnp.maximum(bn, 0.0)
    h2 = jnp.dot(r, w2_ref[...].T, preferred_element_type=jnp.float32)
    g2 = h2 * dinv[:, None]
    g2_ref[0] = g2[:, :HID]
    g2_ref[1] = g2[:, HID:]


def _t3_body(hist_ref, qa_ref, qb_ref, g2_ref, b2_ref, out_ref):
    dinv = _dinv_from_hist(hist_ref[...])
    qa = qa_ref[...]
    qb = qb_ref[...]
    g2 = g2_ref[...]
    oa = (qa[0, :N_NODES] + qa[1, :N_NODES] + g2[0]) * dinv[:, None]
    ob = (qb[0, :N_NODES] + qb[1, :N_NODES] + g2[1]) * dinv[:, None]
    out_ref[...] = jnp.concatenate([oa, ob], axis=1) + b2_ref[...]


def kernel(x, edge_index, W1, b1, gamma, beta, W2, b2):
    e = edge_index.shape[1]
    k0, k1 = _splits(e)
    t = k0 + k1
    e_pad = NS * t * CHUNK
    pad = e_pad - e
    dummy_cols = DUMMY + (jnp.arange(pad, dtype=jnp.int32) % (NROWS - N_NODES))
    rows = jnp.concatenate(
        [edge_index[0], jnp.zeros((pad,), jnp.int32)]).reshape(NS, t, CHUNK)
    cols = jnp.concatenate(
        [edge_index[1], dummy_cols]).reshape(NS, t, CHUNK)
    rows0, rows1 = rows[:, :k0], rows[:, k0:]
    cols0, cols1 = cols[:, :k0], cols[:, k0:]

    hist = _deg_call(k0, k1)(cols0, cols1)

    g1 = pl.pallas_call(
        _t1_body,
        out_shape=jax.ShapeDtypeStruct((N_NODES, HID), jnp.float32),
    )(hist, x, W1)

    acc = _accum_call(HID, k0, k1)
    p1 = acc(g1, rows0, cols0, rows1, cols1)

    g2 = pl.pallas_call(
        _t2_body,
        out_shape=jax.ShapeDtypeStruct((2, N_NODES, HID), jnp.float32),
    )(hist, p1, g1, b1.reshape(1, HID), gamma.reshape(1, HID),
      beta.reshape(1, HID), W2)

    p2a = acc(g2[0], rows0, cols0, rows1, cols1)
    p2b = acc(g2[1], rows0, cols0, rows1, cols1)

    out = pl.pallas_call(
        _t3_body,
        out_shape=jax.ShapeDtypeStruct((N_NODES, OUT_DIM), jnp.float32),
    )(hist, p2a, p2b, g2, b2.reshape(1, OUT_DIM))

    return out


# speedup vs baseline: 2.1223x; 1.8947x over previous
import functools

import jax
import jax.numpy as jnp
from jax import lax
from jax.experimental import pallas as pl
from jax.experimental.pallas import tpu as pltpu
from jax.experimental.pallas import tpu_sc as plsc

N_NODES = 10000
IN_DIM = 128
HID = 64
OUT_DIM = 128

NC = 2
NS = 16
L = 16
CHUNK = 128
NBUF = 3
ROWS_PER_TILE = 640
NROWS = NS * ROWS_PER_TILE
DUMMY = N_NODES
DEGW = 16
CORE_RATIO = 1

_params = pltpu.CompilerParams(use_tc_tiling_on_sc=False)


def _mesh():
    return plsc.VectorSubcoreMesh(core_axis_name="c", subcore_axis_name="s")


def _splits(e):
    total = -(-e // CHUNK)
    t0 = -(-total // NS)
    k1 = max(NBUF, ((t0 + CORE_RATIO) // (CORE_RATIO + 1) + NBUF - 1) // NBUF * NBUF)
    k0 = (max(t0 - k1, NBUF) + NBUF - 1) // NBUF * NBUF
    return k0, k1


def _count(cid, k0, k1):
    return jnp.where(cid == 0, k0, k1)


@functools.lru_cache(maxsize=None)
def _deg_call(k0, k1):
    @functools.partial(
        pl.kernel,
        out_type=jax.ShapeDtypeStruct((NC, NROWS, DEGW), jnp.float32),
        mesh=_mesh(),
        scratch_types=[
            pltpu.VMEM((k0, CHUNK), jnp.int32),
            pltpu.VMEM((CHUNK, DEGW), jnp.float32),
            pltpu.VMEM((CHUNK, DEGW), jnp.float32),
            pltpu.VMEM_SHARED((NROWS, DEGW), jnp.float32),
        ],
        compiler_params=_params,
    )
    def deg_kernel(cols0_hbm, cols1_hbm, out_hbm, colbuf, zbuf, obuf, hist):
        cid = lax.axis_index("c")
        sid = lax.axis_index("s")

        def fill(i, _):
            zbuf[i, :] = jnp.zeros((L,), jnp.float32)
            obuf[i, :] = jnp.full((L,), 1.0, jnp.float32)
            return 0

        lax.fori_loop(0, CHUNK, fill, 0)

        for b in range(ROWS_PER_TILE // CHUNK):
            pltpu.sync_copy(zbuf, hist.at[pl.ds(sid * ROWS_PER_TILE + b * CHUNK, CHUNK)])
        plsc.subcore_barrier()

        @pl.when(cid == 0)
        def _():
            pltpu.sync_copy(cols0_hbm.at[sid], colbuf.at[pl.ds(0, k0)])

        @pl.when(cid == 1)
        def _():
            pltpu.sync_copy(cols1_hbm.at[sid], colbuf.at[pl.ds(0, k1)])

        def step(j, _):
            pltpu.sync_copy(obuf, hist.at[colbuf.at[j]], add=True)
            return 0

        lax.fori_loop(0, _count(cid, k0, k1), step, 0)
        plsc.subcore_barrier()

        for b in range(ROWS_PER_TILE // CHUNK):
            r0 = sid * ROWS_PER_TILE + b * CHUNK
            pltpu.sync_copy(hist.at[pl.ds(r0, CHUNK)], zbuf)
            pltpu.sync_copy(zbuf, out_hbm.at[cid, pl.ds(r0, CHUNK)])

    return deg_kernel


@functools.lru_cache(maxsize=None)
def _accum_call(d, k0, k1):
    kmax = max(k0, k1)

    @functools.partial(
        pl.kernel,
        out_type=jax.ShapeDtypeStruct((NC, NROWS, d), jnp.float32),
        mesh=_mesh(),
        scratch_types=[
            pltpu.VMEM((kmax, CHUNK), jnp.int32),
            pltpu.VMEM((kmax, CHUNK), jnp.int32),
            [pltpu.VMEM((CHUNK, d), jnp.float32) for _ in range(NBUF)],
            pltpu.SemaphoreType.DMA((NBUF,)),
            pltpu.VMEM_SHARED((NROWS, d), jnp.float32),
            pltpu.VMEM_SHARED((NROWS, d), jnp.float32),
        ],
        compiler_params=_params,
    )
    def accum_kernel(g_hbm, rows0_hbm, cols0_hbm, rows1_hbm, cols1_hbm, out_hbm,
                     rowbuf, colbuf, bufs, sem, gtab, accum):
        cid = lax.axis_index("c")
        sid = lax.axis_index("s")
        count = _count(cid, k0, k1)
        buf0 = bufs[0]

        def fill(i, _):
            for k in range(d // L):
                buf0[i, pl.ds(k * L, L)] = jnp.zeros((L,), jnp.float32)
            return 0

        lax.fori_loop(0, CHUNK, fill, 0)

        r_own = pl.ds(sid * ROWS_PER_TILE, ROWS_PER_TILE)
        pltpu.sync_copy(g_hbm.at[r_own], gtab.at[r_own])
        for b in range(ROWS_PER_TILE // CHUNK):
            pltpu.sync_copy(buf0, accum.at[pl.ds(sid * ROWS_PER_TILE + b * CHUNK, CHUNK)])
        plsc.subcore_barrier()

        @pl.when(cid == 0)
        def _():
            pltpu.sync_copy(rows0_hbm.at[sid], rowbuf.at[pl.ds(0, k0)])
            pltpu.sync_copy(cols0_hbm.at[sid], colbuf.at[pl.ds(0, k0)])

        @pl.when(cid == 1)
        def _():
            pltpu.sync_copy(rows1_hbm.at[sid], rowbuf.at[pl.ds(0, k1)])
            pltpu.sync_copy(cols1_hbm.at[sid], colbuf.at[pl.ds(0, k1)])

        for b in range(NBUF):
            pltpu.async_copy(gtab.at[rowbuf.at[b]], bufs[b], sem.at[b])

        def step(t, _):
            for b in range(NBUF):
                j = t * NBUF + b
                pltpu.make_async_copy(gtab.at[rowbuf.at[j]], bufs[b], sem.at[b]).wait()
                pltpu.sync_copy(bufs[b], accum.at[colbuf.at[j]], add=True)

                @pl.when(j + NBUF < count)
                def _():
                    pltpu.async_copy(
                        gtab.at[rowbuf.at[j + NBUF]], bufs[b], sem.at[b])
            return 0

        lax.fori_loop(0, count // NBUF, step, 0)
        plsc.subcore_barrier()

        for b in range(ROWS_PER_TILE // CHUNK):
            r0 = sid * ROWS_PER_TILE + b * CHUNK
            pltpu.sync_copy(accum.at[pl.ds(r0, CHUNK)], buf0)
            pltpu.sync_copy(buf0, out_hbm.at[cid, pl.ds(r0, CHUNK)])

    return accum_kernel


def _dinv_from_hist(hist):
    deg = hist[0, :N_NODES, 0] + hist[1, :N_NODES, 0] + 1.0
    return lax.rsqrt(deg)


def _t1_body(hist_ref, x_ref, w1_ref, g1_ref):
    dinv = _dinv_from_hist(hist_ref[...])
    h = jnp.dot(x_ref[...], w1_ref[...].T, preferred_element_type=jnp.float32)
    g1_ref[...] = jnp.concatenate(
        [h * dinv[:, None], jnp.zeros((NROWS - N_NODES, HID), jnp.float32)], axis=0)


def _t2_body(hist_ref, p_ref, g1_ref, b1_ref, gamma_ref, beta_ref, w2_ref, g2_ref):
    dinv = _dinv_from_hist(hist_ref[...])
    p = p_ref[...]
    a = (p[0, :N_NODES] + p[1, :N_NODES] + g1_ref[:N_NODES]) * dinv[:, None] + b1_ref[...]
    mean = jnp.mean(a, axis=0)
    var = jnp.mean((a - mean) ** 2, axis=0)
    bn = (a - mean) * lax.rsqrt(var + 1e-5) * gamma_ref[...] + beta_ref[...]
    r = jnp.maximum(bn, 0.0)
    h2 = jnp.dot(r, w2_ref[...].T, preferred_element_type=jnp.float32)
    g2 = h2 * dinv[:, None]
    z = jnp.zeros((NROWS - N_NODES, HID), jnp.float32)
    g2_ref[0] = jnp.concatenate([g2[:, :HID], z], axis=0)
    g2_ref[1] = jnp.concatenate([g2[:, HID:], z], axis=0)


def _t3_body(hist_ref, qa_ref, qb_ref, g2_ref, b2_ref, out_ref):
    dinv = _dinv_from_hist(hist_ref[...])
    qa = qa_ref[...]
    qb = qb_ref[...]
    g2 = g2_ref[...]
    oa = (qa[0, :N_NODES] + qa[1, :N_NODES] + g2[0, :N_NODES]) * dinv[:, None]
    ob = (qb[0, :N_NODES] + qb[1, :N_NODES] + g2[1, :N_NODES]) * dinv[:, None]
    out_ref[...] = jnp.concatenate([oa, ob], axis=1) + b2_ref[...]


def kernel(x, edge_index, W1, b1, gamma, beta, W2, b2):
    e = edge_index.shape[1]
    k0, k1 = _splits(e)
    t = k0 + k1
    e_pad = NS * t * CHUNK
    pad = e_pad - e
    dummy_cols = DUMMY + (jnp.arange(pad, dtype=jnp.int32) % (NROWS - N_NODES))
    rows = jnp.concatenate(
        [edge_index[0], jnp.zeros((pad,), jnp.int32)]).reshape(NS, t, CHUNK)
    cols = jnp.concatenate(
        [edge_index[1], dummy_cols]).reshape(NS, t, CHUNK)
    rows0, rows1 = rows[:, :k0], rows[:, k0:]
    cols0, cols1 = cols[:, :k0], cols[:, k0:]

    hist = _deg_call(k0, k1)(cols0, cols1)

    g1 = pl.pallas_call(
        _t1_body,
        out_shape=jax.ShapeDtypeStruct((NROWS, HID), jnp.float32),
    )(hist, x, W1)

    acc = _accum_call(HID, k0, k1)
    p1 = acc(g1, rows0, cols0, rows1, cols1)

    g2 = pl.pallas_call(
        _t2_body,
        out_shape=jax.ShapeDtypeStruct((2, NROWS, HID), jnp.float32),
    )(hist, p1, g1, b1.reshape(1, HID), gamma.reshape(1, HID),
      beta.reshape(1, HID), W2)

    p2a = acc(g2[0], rows0, cols0, rows1, cols1)
    p2b = acc(g2[1], rows0, cols0, rows1, cols1)

    out = pl.pallas_call(
        _t3_body,
        out_shape=jax.ShapeDtypeStruct((N_NODES, OUT_DIM), jnp.float32),
    )(hist, p2a, p2b, g2, b2.reshape(1, OUT_DIM))

    return out


# speedup vs baseline: 2.1418x; 1.0092x over previous
import functools

import jax
import jax.numpy as jnp
from jax import lax
from jax.experimental import pallas as pl
from jax.experimental.pallas import tpu as pltpu
from jax.experimental.pallas import tpu_sc as plsc

N_NODES = 10000
IN_DIM = 128
HID = 64
OUT_DIM = 128

NC = 2
NS = 16
L = 16
CHUNK = 128
NBUF = 3
ROWS_PER_TILE = 640
NROWS = NS * ROWS_PER_TILE
DUMMY = N_NODES
DEGW = 16
CORE_RATIO = 1

_params = pltpu.CompilerParams(use_tc_tiling_on_sc=False)


def _mesh():
    return plsc.VectorSubcoreMesh(core_axis_name="c", subcore_axis_name="s")


def _splits(e):
    total = -(-e // CHUNK)
    t0 = -(-total // NS)
    k1 = max(NBUF, ((t0 + CORE_RATIO) // (CORE_RATIO + 1) + NBUF - 1) // NBUF * NBUF)
    k0 = (max(t0 - k1, NBUF) + NBUF - 1) // NBUF * NBUF
    return k0, k1


def _count(cid, k0, k1):
    return jnp.where(cid == 0, k0, k1)


@functools.lru_cache(maxsize=None)
def _deg_call(k0, k1):
    @functools.partial(
        pl.kernel,
        out_type=jax.ShapeDtypeStruct((NC, NROWS, DEGW), jnp.float32),
        mesh=_mesh(),
        scratch_types=[
            pltpu.VMEM((k0, CHUNK), jnp.int32),
            pltpu.VMEM((CHUNK, DEGW), jnp.float32),
            pltpu.VMEM((CHUNK, DEGW), jnp.float32),
            pltpu.VMEM_SHARED((NROWS, DEGW), jnp.float32),
        ],
        compiler_params=_params,
    )
    def deg_kernel(cols0_hbm, cols1_hbm, out_hbm, colbuf, zbuf, obuf, hist):
        cid = lax.axis_index("c")
        sid = lax.axis_index("s")

        def fill(i, _):
            zbuf[i, :] = jnp.zeros((L,), jnp.float32)
            obuf[i, :] = jnp.full((L,), 1.0, jnp.float32)
            return 0

        lax.fori_loop(0, CHUNK, fill, 0)

        for b in range(ROWS_PER_TILE // CHUNK):
            pltpu.sync_copy(zbuf, hist.at[pl.ds(sid * ROWS_PER_TILE + b * CHUNK, CHUNK)])
        plsc.subcore_barrier()

        @pl.when(cid == 0)
        def _():
            pltpu.sync_copy(cols0_hbm.at[sid], colbuf.at[pl.ds(0, k0)])

        @pl.when(cid == 1)
        def _():
            pltpu.sync_copy(cols1_hbm.at[sid], colbuf.at[pl.ds(0, k1)])

        def step(j, _):
            pltpu.sync_copy(obuf, hist.at[colbuf.at[j]], add=True)
            return 0

        lax.fori_loop(0, _count(cid, k0, k1), step, 0)
        plsc.subcore_barrier()

        for b in range(ROWS_PER_TILE // CHUNK):
            r0 = sid * ROWS_PER_TILE + b * CHUNK
            pltpu.sync_copy(hist.at[pl.ds(r0, CHUNK)], zbuf)
            pltpu.sync_copy(zbuf, out_hbm.at[cid, pl.ds(r0, CHUNK)])

    return deg_kernel


@functools.lru_cache(maxsize=None)
def _accum_call(d, k0, k1):
    kmax = max(k0, k1)

    @functools.partial(
        pl.kernel,
        out_type=jax.ShapeDtypeStruct((NC, NROWS, d), jnp.float32),
        mesh=_mesh(),
        scratch_types=[
            pltpu.VMEM((kmax, CHUNK), jnp.int32),
            pltpu.VMEM((kmax, CHUNK), jnp.int32),
            [pltpu.VMEM((CHUNK, d), jnp.float32) for _ in range(NBUF)],
            pltpu.SemaphoreType.DMA((NBUF,)),
            pltpu.VMEM_SHARED((NROWS, d), jnp.float32),
            pltpu.VMEM_SHARED((NROWS, d), jnp.float32),
        ],
        compiler_params=_params,
    )
    def accum_kernel(g_hbm, rows0_hbm, cols0_hbm, rows1_hbm, cols1_hbm, out_hbm,
                     rowbuf, colbuf, bufs, sem, gtab, accum):
        cid = lax.axis_index("c")
        sid = lax.axis_index("s")
        count = _count(cid, k0, k1)
        buf0 = bufs[0]

        def fill(i, _):
            for k in range(d // L):
                buf0[i, pl.ds(k * L, L)] = jnp.zeros((L,), jnp.float32)
            return 0

        lax.fori_loop(0, CHUNK, fill, 0)

        r_own = pl.ds(sid * ROWS_PER_TILE, ROWS_PER_TILE)
        pltpu.sync_copy(g_hbm.at[r_own], gtab.at[r_own])
        for b in range(ROWS_PER_TILE // CHUNK):
            pltpu.sync_copy(buf0, accum.at[pl.ds(sid * ROWS_PER_TILE + b * CHUNK, CHUNK)])
        plsc.subcore_barrier()

        @pl.when(cid == 0)
        def _():
            pltpu.sync_copy(rows0_hbm.at[sid], rowbuf.at[pl.ds(0, k0)])
            pltpu.sync_copy(cols0_hbm.at[sid], colbuf.at[pl.ds(0, k0)])

        @pl.when(cid == 1)
        def _():
            pltpu.sync_copy(rows1_hbm.at[sid], rowbuf.at[pl.ds(0, k1)])
            pltpu.sync_copy(cols1_hbm.at[sid], colbuf.at[pl.ds(0, k1)])

        for b in range(NBUF):
            pltpu.async_copy(gtab.at[rowbuf.at[b]], bufs[b], sem.at[b])

        def step(t, _):
            for b in range(NBUF):
                j = t * NBUF + b
                pltpu.make_async_copy(gtab.at[rowbuf.at[j]], bufs[b], sem.at[b]).wait()
                pltpu.sync_copy(bufs[b], accum.at[colbuf.at[j]], add=True)

                @pl.when(j + NBUF < count)
                def _():
                    pltpu.async_copy(
                        gtab.at[rowbuf.at[j + NBUF]], bufs[b], sem.at[b])
            return 0

        lax.fori_loop(0, count // NBUF, step, 0)
        plsc.subcore_barrier()

        for b in range(ROWS_PER_TILE // CHUNK):
            r0 = sid * ROWS_PER_TILE + b * CHUNK
            pltpu.sync_copy(accum.at[pl.ds(r0, CHUNK)], buf0)
            pltpu.sync_copy(buf0, out_hbm.at[cid, pl.ds(r0, CHUNK)])

    return accum_kernel


def _dinv_from_hist(hist):
    deg = hist[0, :N_NODES, 0] + hist[1, :N_NODES, 0] + 1.0
    return lax.rsqrt(deg)


def _t1_body(hist_ref, x_ref, w1_ref, g1_ref, dinv_ref):
    dinv = _dinv_from_hist(hist_ref[...])
    h = jnp.dot(x_ref[...], w1_ref[...].T, preferred_element_type=jnp.float32)
    g1_ref[...] = jnp.concatenate(
        [h * dinv[:, None], jnp.zeros((NROWS - N_NODES, HID), jnp.float32)], axis=0)
    dinv_ref[...] = dinv[:, None]


def _t2_body(dinv_ref, p_ref, g1_ref, b1_ref, gamma_ref, beta_ref, w2_ref, g2_ref):
    dinv = dinv_ref[...]
    p = p_ref[...]
    a = (p[0, :N_NODES] + p[1, :N_NODES] + g1_ref[:N_NODES]) * dinv + b1_ref[...]
    mean = jnp.mean(a, axis=0)
    var = jnp.mean((a - mean) ** 2, axis=0)
    bn = (a - mean) * lax.rsqrt(var + 1e-5) * gamma_ref[...] + beta_ref[...]
    r = jnp.maximum(bn, 0.0)
    h2 = jnp.dot(r, w2_ref[...].T, preferred_element_type=jnp.float32)
    g2 = h2 * dinv
    z = jnp.zeros((NROWS - N_NODES, HID), jnp.float32)
    g2_ref[0] = jnp.concatenate([g2[:, :HID], z], axis=0)
    g2_ref[1] = jnp.concatenate([g2[:, HID:], z], axis=0)


def _t3_body(dinv_ref, qa_ref, qb_ref, g2_ref, b2_ref, out_ref):
    dinv = dinv_ref[...]
    qa = qa_ref[...]
    qb = qb_ref[...]
    g2 = g2_ref[...]
    oa = (qa[0] + qa[1] + g2[0]) * dinv
    ob = (qb[0] + qb[1] + g2[1]) * dinv
    out_ref[...] = jnp.concatenate([oa, ob], axis=1) + b2_ref[...]


def kernel(x, edge_index, W1, b1, gamma, beta, W2, b2):
    e = edge_index.shape[1]
    k0, k1 = _splits(e)
    t = k0 + k1
    e_pad = NS * t * CHUNK
    pad = e_pad - e
    dummy_cols = DUMMY + (jnp.arange(pad, dtype=jnp.int32) % (NROWS - N_NODES))
    rows = jnp.concatenate(
        [edge_index[0], jnp.zeros((pad,), jnp.int32)]).reshape(NS, t, CHUNK)
    cols = jnp.concatenate(
        [edge_index[1], dummy_cols]).reshape(NS, t, CHUNK)
    rows0, rows1 = rows[:, :k0], rows[:, k0:]
    cols0, cols1 = cols[:, :k0], cols[:, k0:]

    hist = _deg_call(k0, k1)(cols0, cols1)

    g1, dinv = pl.pallas_call(
        _t1_body,
        out_shape=(jax.ShapeDtypeStruct((NROWS, HID), jnp.float32),
                   jax.ShapeDtypeStruct((N_NODES, 1), jnp.float32)),
    )(hist, x, W1)

    acc = _accum_call(HID, k0, k1)
    p1 = acc(g1, rows0, cols0, rows1, cols1)

    g2 = pl.pallas_call(
        _t2_body,
        out_shape=jax.ShapeDtypeStruct((2, NROWS, HID), jnp.float32),
    )(dinv, p1, g1, b1.reshape(1, HID), gamma.reshape(1, HID),
      beta.reshape(1, HID), W2)

    p2a = acc(g2[0], rows0, cols0, rows1, cols1)
    p2b = acc(g2[1], rows0, cols0, rows1, cols1)

    blk = 1000
    grid = N_NODES // blk
    out = pl.pallas_call(
        _t3_body,
        grid=(grid,),
        in_specs=[
            pl.BlockSpec((blk, 1), lambda i: (i, 0)),
            pl.BlockSpec((NC, blk, HID), lambda i: (0, i, 0)),
            pl.BlockSpec((NC, blk, HID), lambda i: (0, i, 0)),
            pl.BlockSpec((2, blk, HID), lambda i: (0, i, 0)),
            pl.BlockSpec((1, OUT_DIM), lambda i: (0, 0)),
        ],
        out_specs=pl.BlockSpec((blk, OUT_DIM), lambda i: (i, 0)),
        out_shape=jax.ShapeDtypeStruct((N_NODES, OUT_DIM), jnp.float32),
    )(dinv, p2a, p2b, g2, b2.reshape(1, OUT_DIM))

    return out
